# baseline (speedup 1.0000x reference)
import functools

import jax
import jax.numpy as jnp
from jax import lax
from jax.experimental import pallas as pl
from jax.experimental.pallas import tpu as pltpu
from jax.experimental.pallas import tpu_sc as plsc

_B = 16
_NH = 1024
_D = 128
_E = 262144
_NC = 2
_NS = 16
_LN = 16

_EW = _E // _NS
_ROWS = _EW // 128
_GW = _NH * _NH
_HGW = _GW // 2
_NP = _B * 2 // _NC
_SW = _HGW // _NS
_ZCH = 16384
_PAD = _HGW


def _sc_scatter_body(start_hbm, end_hbm, a_hbm,
                     start_v, end_v, idx_v, pk_v, zf_v, ones_v, smem_s,
                     esem, zsem, ssem, csem):
    cid = lax.axis_index("c")
    sid = lax.axis_index("s")
    ebase = sid * _EW

    e1 = pltpu.async_copy(start_hbm.at[pl.ds(ebase, _EW)], start_v, esem)
    e2 = pltpu.async_copy(end_hbm.at[pl.ds(ebase, _EW)], end_v, esem)

    with jax.named_scope("zfill"):
        @plsc.parallel_loop(0, _ZCH, _LN, unroll=8)
        def _zfill(i):
            zf_v[pl.ds(i, _LN)] = jnp.zeros((_LN,), jnp.float32)
        for k in range(128 // _LN):
            ones_v[pl.ds(k * _LN, _LN)] = jnp.ones((_LN,), jnp.float32)

    e1.wait()
    e2.wait()

    with jax.named_scope("pack"):
        @plsc.parallel_loop(0, _EW, _LN, unroll=8)
        def _pk(i):
            s = start_v[pl.ds(i, _LN)]
            e = end_v[pl.ds(i, _LN)]
            ge = lax.shift_right_logical(s, 10)
            local = lax.shift_left(jnp.bitwise_and(s, 1023), 10) \
                + jnp.bitwise_and(e, 1023)
            pk_v[pl.ds(i, _LN)] = jnp.bitwise_or(lax.shift_left(ge, 20), local)

    dummy = _PAD + sid * _LN + lax.iota(jnp.int32, _LN)

    for p in range(_NP):
        g = cid * (_NP // 2) + (p // 2)
        half = p % 2
        key = g * 2 + half

        with jax.named_scope("copy_wait"):
            if p > 0:
                pltpu.make_async_copy(
                    smem_s.at[pl.ds(sid * _SW, _SW)],
                    a_hbm.at[pl.ds(0, _SW)], csem).wait()

        with jax.named_scope("zero_fire"):
            zh = [pltpu.async_copy(
                      zf_v, smem_s.at[pl.ds(sid * _SW + j * _ZCH, _ZCH)], zsem)
                  for j in range(_SW // _ZCH)]

        with jax.named_scope("idx_compute"):
            @plsc.parallel_loop(0, _EW, _LN, unroll=8)
            def _ib(i):
                pk = pk_v[pl.ds(i, _LN)]
                mine = lax.shift_right_logical(pk, 19) == key
                local = jnp.bitwise_and(pk, _HGW - 1)
                idx_v[lax.div(i, 128), pl.ds(lax.rem(i, 128), _LN)] = \
                    jnp.where(mine, local, dummy)

        with jax.named_scope("zero_drain"):
            for h in zh:
                h.wait()
        plsc.subcore_barrier()

        with jax.named_scope("scatter"):
            @plsc.parallel_loop(0, _ROWS, 1, unroll=8)
            def _sb(j):
                pltpu.async_copy(ones_v, smem_s.at[idx_v.at[j]], ssem)
            pltpu.make_async_copy(a_hbm.at[pl.ds(0, _ZCH)], zf_v, ssem).wait()
        plsc.subcore_barrier()

        with jax.named_scope("copy_out"):
            pltpu.async_copy(
                smem_s.at[pl.ds(sid * _SW, _SW)],
                a_hbm.at[pl.ds(g * _GW + half * _HGW + sid * _SW, _SW)], csem)

    with jax.named_scope("final_wait"):
        pltpu.make_async_copy(
            smem_s.at[pl.ds(sid * _SW, _SW)],
            a_hbm.at[pl.ds(0, _SW)], csem).wait()


_scatter_adj = functools.partial(
    pl.kernel,
    out_type=jax.ShapeDtypeStruct((_B * _NH * _NH,), jnp.float32),
    mesh=plsc.VectorSubcoreMesh(core_axis_name="c", subcore_axis_name="s"),
    scratch_types=[
        pltpu.VMEM((_EW,), jnp.int32),
        pltpu.VMEM((_EW,), jnp.int32),
        pltpu.VMEM((_ROWS, 128), jnp.int32),
        pltpu.VMEM((_EW,), jnp.int32),
        pltpu.VMEM((_ZCH,), jnp.float32),
        pltpu.VMEM((128,), jnp.float32),
        pltpu.VMEM_SHARED((_HGW + 512,), jnp.float32),
        pltpu.SemaphoreType.DMA,
        pltpu.SemaphoreType.DMA,
        pltpu.SemaphoreType.DMA,
        pltpu.SemaphoreType.DMA,
    ],
)(_sc_scatter_body)


def _tc_body(a_ref, x_ref, w_ref, bias_ref, w1_ref, b1_ref, w2_ref, b2_ref,
             w3_ref, b3_ref, out_ref, pool_scr):
    bi = pl.program_id(0)
    y = jnp.dot(a_ref[...], x_ref[...], preferred_element_type=jnp.float32)
    z = jnp.dot(y, w_ref[...], preferred_element_type=jnp.float32) + bias_ref[...]
    pool_scr[pl.ds(bi, 1), :] = jnp.max(z, axis=0, keepdims=True)

    @pl.when(bi == _B - 1)
    def _mlp():
        p = pool_scr[...]
        h = jnp.tanh(jnp.dot(p, w1_ref[...], preferred_element_type=jnp.float32)
                     + b1_ref[...])
        h = jnp.tanh(jnp.dot(h, w2_ref[...], preferred_element_type=jnp.float32)
                     + b2_ref[...])
        out_ref[...] = (jnp.dot(h, w3_ref[...], preferred_element_type=jnp.float32)
                        + b3_ref[...])


_gcn_head = pl.pallas_call(
    _tc_body,
    grid=(_B,),
    in_specs=[
        pl.BlockSpec((None, _NH, _NH), lambda b: (b, 0, 0)),
        pl.BlockSpec((None, _NH, _D), lambda b: (b, 0, 0)),
        pl.BlockSpec((_D, _D), lambda b: (0, 0)),
        pl.BlockSpec((1, _D), lambda b: (0, 0)),
        pl.BlockSpec((_D, _D), lambda b: (0, 0)),
        pl.BlockSpec((1, _D), lambda b: (0, 0)),
        pl.BlockSpec((_D, _D), lambda b: (0, 0)),
        pl.BlockSpec((1, _D), lambda b: (0, 0)),
        pl.BlockSpec((_D, _D), lambda b: (0, 0)),
        pl.BlockSpec((1, _D), lambda b: (0, 0)),
    ],
    out_specs=pl.BlockSpec((_B, _D), lambda b: (0, 0)),
    out_shape=jax.ShapeDtypeStruct((_B, _D), jnp.float32),
    scratch_shapes=[pltpu.VMEM((_B, _D), jnp.float32)],
)


def kernel(x, edge_index, batch, batch_size, W, b, W1, b1, W2, b2, W3, b3):
    N, d = x.shape
    start = edge_index[0]
    end = edge_index[1]
    a_flat = _scatter_adj(start, end)
    a3 = a_flat.reshape(_B, _NH, _NH)
    nodes = x.reshape(_B, _NH, d)
    w3p = jnp.pad(W3, ((0, 0), (0, _D - W3.shape[1])))
    b3p = jnp.pad(b3, (0, _D - b3.shape[0])).reshape(1, _D)
    out_full = _gcn_head(a3, nodes, W, b.reshape(1, _D), W1, b1.reshape(1, _D),
                         W2, b2.reshape(1, _D), w3p, b3p)
    return out_full[:, :1]


# speedup vs baseline: 1.1398x; 1.1398x over previous
import functools

import jax
import jax.numpy as jnp
from jax import lax
from jax.experimental import pallas as pl
from jax.experimental.pallas import tpu as pltpu
from jax.experimental.pallas import tpu_sc as plsc

_B = 16
_NH = 1024
_D = 128
_E = 262144
_NC = 2
_NS = 16
_LN = 16
_GB = _B // 2

_EW = _E // _NS
_ROWS = _EW // 128
_GW = _NH * _NH
_HGW = _GW // 2
_NP = _GB * 2 // _NC
_SW = _HGW // _NS
_ZCH = 16384
_PAD = _HGW


def _make_scatter(gbase):
    def _body(start_hbm, end_hbm, a_hbm,
              start_v, end_v, idx_v, pk_v, zf_v, ones_v, smem_s,
              esem, zsem, ssem, csem):
        cid = lax.axis_index("c")
        sid = lax.axis_index("s")
        ebase = sid * _EW

        e1 = pltpu.async_copy(start_hbm.at[pl.ds(ebase, _EW)], start_v, esem)
        e2 = pltpu.async_copy(end_hbm.at[pl.ds(ebase, _EW)], end_v, esem)

        with jax.named_scope("zfill"):
            @plsc.parallel_loop(0, _ZCH, _LN, unroll=8)
            def _zfill(i):
                zf_v[pl.ds(i, _LN)] = jnp.zeros((_LN,), jnp.float32)
            for k in range(128 // _LN):
                ones_v[pl.ds(k * _LN, _LN)] = jnp.ones((_LN,), jnp.float32)

        e1.wait()
        e2.wait()

        with jax.named_scope("pack"):
            @plsc.parallel_loop(0, _EW, _LN, unroll=8)
            def _pk(i):
                s = start_v[pl.ds(i, _LN)]
                e = end_v[pl.ds(i, _LN)]
                ge = lax.shift_right_logical(s, 10)
                local = lax.shift_left(jnp.bitwise_and(s, 1023), 10) \
                    + jnp.bitwise_and(e, 1023)
                pk_v[pl.ds(i, _LN)] = jnp.bitwise_or(lax.shift_left(ge, 20),
                                                     local)

        dummy = _PAD + sid * _LN + lax.iota(jnp.int32, _LN)

        for p in range(_NP):
            g = gbase + cid * (_NP // 2) + (p // 2)
            half = p % 2
            key = g * 2 + half

            with jax.named_scope("copy_wait"):
                if p > 0:
                    pltpu.make_async_copy(
                        smem_s.at[pl.ds(sid * _SW, _SW)],
                        a_hbm.at[pl.ds(0, _SW)], csem).wait()

            with jax.named_scope("zero_fire"):
                zh = [pltpu.async_copy(
                          zf_v,
                          smem_s.at[pl.ds(sid * _SW + j * _ZCH, _ZCH)], zsem)
                      for j in range(_SW // _ZCH)]

            with jax.named_scope("idx_compute"):
                @plsc.parallel_loop(0, _EW, _LN, unroll=8)
                def _ib(i):
                    pk = pk_v[pl.ds(i, _LN)]
                    mine = lax.shift_right_logical(pk, 19) == key
                    local = jnp.bitwise_and(pk, _HGW - 1)
                    idx_v[lax.div(i, 128), pl.ds(lax.rem(i, 128), _LN)] = \
                        jnp.where(mine, local, dummy)

            with jax.named_scope("zero_drain"):
                for h in zh:
                    h.wait()
            plsc.subcore_barrier()

            with jax.named_scope("scatter"):
                @plsc.parallel_loop(0, _ROWS, 1, unroll=8)
                def _sb(j):
                    pltpu.async_copy(ones_v, smem_s.at[idx_v.at[j]], ssem)
                pltpu.make_async_copy(a_hbm.at[pl.ds(0, _ZCH)], zf_v,
                                      ssem).wait()
            plsc.subcore_barrier()

            with jax.named_scope("copy_out"):
                pltpu.async_copy(
                    smem_s.at[pl.ds(sid * _SW, _SW)],
                    a_hbm.at[pl.ds((g - gbase) * _GW + half * _HGW
                                   + sid * _SW, _SW)], csem)

        with jax.named_scope("final_wait"):
            pltpu.make_async_copy(
                smem_s.at[pl.ds(sid * _SW, _SW)],
                a_hbm.at[pl.ds(0, _SW)], csem).wait()

    return functools.partial(
        pl.kernel,
        out_type=jax.ShapeDtypeStruct((_GB * _NH * _NH,), jnp.float32),
        mesh=plsc.VectorSubcoreMesh(core_axis_name="c", subcore_axis_name="s"),
        scratch_types=[
            pltpu.VMEM((_EW,), jnp.int32),
            pltpu.VMEM((_EW,), jnp.int32),
            pltpu.VMEM((_ROWS, 128), jnp.int32),
            pltpu.VMEM((_EW,), jnp.int32),
            pltpu.VMEM((_ZCH,), jnp.float32),
            pltpu.VMEM((128,), jnp.float32),
            pltpu.VMEM_SHARED((_HGW + 512,), jnp.float32),
            pltpu.SemaphoreType.DMA,
            pltpu.SemaphoreType.DMA,
            pltpu.SemaphoreType.DMA,
            pltpu.SemaphoreType.DMA,
        ],
    )(_body)


_scatter_lo = _make_scatter(0)
_scatter_hi = _make_scatter(_GB)


def _tc_pool_body(a_ref, x_ref, w_ref, bias_ref, out_ref):
    bi = pl.program_id(0)
    y = jnp.dot(a_ref[...], x_ref[...], preferred_element_type=jnp.float32)
    z = jnp.dot(y, w_ref[...], preferred_element_type=jnp.float32) \
        + bias_ref[...]
    out_ref[pl.ds(bi, 1), :] = jnp.max(z, axis=0, keepdims=True)


def _tc_pool_mlp_body(a_ref, x_ref, w_ref, bias_ref, p0_ref, w1_ref, b1_ref,
                      w2_ref, b2_ref, w3_ref, b3_ref, out_ref, pool_scr):
    bi = pl.program_id(0)
    y = jnp.dot(a_ref[...], x_ref[...], preferred_element_type=jnp.float32)
    z = jnp.dot(y, w_ref[...], preferred_element_type=jnp.float32) \
        + bias_ref[...]
    pool_scr[pl.ds(bi, 1), :] = jnp.max(z, axis=0, keepdims=True)

    @pl.when(bi == _GB - 1)
    def _mlp():
        p = jnp.concatenate([p0_ref[...], pool_scr[...]], axis=0)
        h = jnp.tanh(jnp.dot(p, w1_ref[...], preferred_element_type=jnp.float32)
                     + b1_ref[...])
        h = jnp.tanh(jnp.dot(h, w2_ref[...], preferred_element_type=jnp.float32)
                     + b2_ref[...])
        out_ref[...] = (jnp.dot(h, w3_ref[...],
                                preferred_element_type=jnp.float32)
                        + b3_ref[...])


_W2D = pl.BlockSpec((_D, _D), lambda b: (0, 0))
_B2D = pl.BlockSpec((1, _D), lambda b: (0, 0))

_gcn_pool = pl.pallas_call(
    _tc_pool_body,
    grid=(_GB,),
    in_specs=[
        pl.BlockSpec((None, _NH, _NH), lambda b: (b, 0, 0)),
        pl.BlockSpec((None, _NH, _D), lambda b: (b, 0, 0)),
        _W2D, _B2D,
    ],
    out_specs=pl.BlockSpec((_GB, _D), lambda b: (0, 0)),
    out_shape=jax.ShapeDtypeStruct((_GB, _D), jnp.float32),
)

_gcn_pool_mlp = pl.pallas_call(
    _tc_pool_mlp_body,
    grid=(_GB,),
    in_specs=[
        pl.BlockSpec((None, _NH, _NH), lambda b: (b, 0, 0)),
        pl.BlockSpec((None, _NH, _D), lambda b: (b, 0, 0)),
        _W2D, _B2D,
        pl.BlockSpec((_GB, _D), lambda b: (0, 0)),
        _W2D, _B2D, _W2D, _B2D, _W2D, _B2D,
    ],
    out_specs=pl.BlockSpec((_B, _D), lambda b: (0, 0)),
    out_shape=jax.ShapeDtypeStruct((_B, _D), jnp.float32),
    scratch_shapes=[pltpu.VMEM((_GB, _D), jnp.float32)],
)


def kernel(x, edge_index, batch, batch_size, W, b, W1, b1, W2, b2, W3, b3):
    N, d = x.shape
    start = edge_index[0]
    end = edge_index[1]
    a_lo = _scatter_lo(start, end).reshape(_GB, _NH, _NH)
    a_hi = _scatter_hi(start, end).reshape(_GB, _NH, _NH)
    nodes = x.reshape(_B, _NH, d)
    bias2 = b.reshape(1, _D)
    w3p = jnp.pad(W3, ((0, 0), (0, _D - W3.shape[1])))
    b3p = jnp.pad(b3, (0, _D - b3.shape[0])).reshape(1, _D)
    pooled_lo = _gcn_pool(a_lo, nodes[:_GB], W, bias2)
    out_full = _gcn_pool_mlp(a_hi, nodes[_GB:], W, bias2, pooled_lo,
                             W1, b1.reshape(1, _D), W2, b2.reshape(1, _D),
                             w3p, b3p)
    return out_full[:, :1]


# speedup vs baseline: 1.1605x; 1.0182x over previous
import functools

import jax
import jax.numpy as jnp
from jax import lax
from jax.experimental import pallas as pl
from jax.experimental.pallas import tpu as pltpu
from jax.experimental.pallas import tpu_sc as plsc

_B = 16
_NH = 1024
_D = 128
_E = 262144
_NC = 2
_NS = 16
_LN = 16
_GB = _B // 4

_EW = _E // _NS
_ROWS = _EW // 128
_GW = _NH * _NH
_HGW = _GW // 2
_NP = _GB * 2 // _NC
_SW = _HGW // _NS
_ZCH = 16384
_PAD = _HGW


def _make_scatter(gbase):
    def _body(start_hbm, end_hbm, a_hbm,
              start_v, end_v, idx_v, pk_v, zf_v, ones_v, smem_s,
              esem, zsem, ssem, csem):
        cid = lax.axis_index("c")
        sid = lax.axis_index("s")
        ebase = sid * _EW

        e1 = pltpu.async_copy(start_hbm.at[pl.ds(ebase, _EW)], start_v, esem)
        e2 = pltpu.async_copy(end_hbm.at[pl.ds(ebase, _EW)], end_v, esem)

        with jax.named_scope("zfill"):
            @plsc.parallel_loop(0, _ZCH, _LN, unroll=8)
            def _zfill(i):
                zf_v[pl.ds(i, _LN)] = jnp.zeros((_LN,), jnp.float32)
            for k in range(128 // _LN):
                ones_v[pl.ds(k * _LN, _LN)] = jnp.ones((_LN,), jnp.float32)

        e1.wait()
        e2.wait()

        with jax.named_scope("pack"):
            @plsc.parallel_loop(0, _EW, _LN, unroll=8)
            def _pk(i):
                s = start_v[pl.ds(i, _LN)]
                e = end_v[pl.ds(i, _LN)]
                ge = lax.shift_right_logical(s, 10)
                local = lax.shift_left(jnp.bitwise_and(s, 1023), 10) \
                    + jnp.bitwise_and(e, 1023)
                pk_v[pl.ds(i, _LN)] = jnp.bitwise_or(lax.shift_left(ge, 20),
                                                     local)

        dummy = _PAD + sid * _LN + lax.iota(jnp.int32, _LN)

        for p in range(_NP):
            g = gbase + cid * (_NP // 2) + (p // 2)
            half = p % 2
            key = g * 2 + half

            with jax.named_scope("copy_wait"):
                if p > 0:
                    pltpu.make_async_copy(
                        smem_s.at[pl.ds(sid * _SW, _SW)],
                        a_hbm.at[pl.ds(0, _SW)], csem).wait()

            with jax.named_scope("zero_fire"):
                zh = [pltpu.async_copy(
                          zf_v,
                          smem_s.at[pl.ds(sid * _SW + j * _ZCH, _ZCH)], zsem)
                      for j in range(_SW // _ZCH)]

            with jax.named_scope("idx_compute"):
                @plsc.parallel_loop(0, _EW, _LN, unroll=8)
                def _ib(i):
                    pk = pk_v[pl.ds(i, _LN)]
                    mine = lax.shift_right_logical(pk, 19) == key
                    local = jnp.bitwise_and(pk, _HGW - 1)
                    idx_v[lax.div(i, 128), pl.ds(lax.rem(i, 128), _LN)] = \
                        jnp.where(mine, local, dummy)

            with jax.named_scope("zero_drain"):
                for h in zh:
                    h.wait()
            plsc.subcore_barrier()

            with jax.named_scope("scatter"):
                @plsc.parallel_loop(0, _ROWS, 1, unroll=8)
                def _sb(j):
                    pltpu.async_copy(ones_v, smem_s.at[idx_v.at[j]], ssem)
                pltpu.make_async_copy(a_hbm.at[pl.ds(0, _ZCH)], zf_v,
                                      ssem).wait()
            plsc.subcore_barrier()

            with jax.named_scope("copy_out"):
                pltpu.async_copy(
                    smem_s.at[pl.ds(sid * _SW, _SW)],
                    a_hbm.at[pl.ds((g - gbase) * _GW + half * _HGW
                                   + sid * _SW, _SW)], csem)

        with jax.named_scope("final_wait"):
            pltpu.make_async_copy(
                smem_s.at[pl.ds(sid * _SW, _SW)],
                a_hbm.at[pl.ds(0, _SW)], csem).wait()

    return functools.partial(
        pl.kernel,
        out_type=jax.ShapeDtypeStruct((_GB * _NH * _NH,), jnp.float32),
        mesh=plsc.VectorSubcoreMesh(core_axis_name="c", subcore_axis_name="s"),
        scratch_types=[
            pltpu.VMEM((_EW,), jnp.int32),
            pltpu.VMEM((_EW,), jnp.int32),
            pltpu.VMEM((_ROWS, 128), jnp.int32),
            pltpu.VMEM((_EW,), jnp.int32),
            pltpu.VMEM((_ZCH,), jnp.float32),
            pltpu.VMEM((128,), jnp.float32),
            pltpu.VMEM_SHARED((_HGW + 512,), jnp.float32),
            pltpu.SemaphoreType.DMA,
            pltpu.SemaphoreType.DMA,
            pltpu.SemaphoreType.DMA,
            pltpu.SemaphoreType.DMA,
        ],
    )(_body)


_scatter_chunks = [_make_scatter(gb) for gb in range(0, _B, _GB)]


def _tc_pool_body(a_ref, x_ref, w_ref, bias_ref, out_ref):
    bi = pl.program_id(0)
    y = jnp.dot(a_ref[...], x_ref[...], preferred_element_type=jnp.float32)
    z = jnp.dot(y, w_ref[...], preferred_element_type=jnp.float32) \
        + bias_ref[...]
    out_ref[pl.ds(bi, 1), :] = jnp.max(z, axis=0, keepdims=True)


def _tc_pool_mlp_body(a_ref, x_ref, w_ref, bias_ref, p0_ref, p1_ref, p2_ref,
                      w1_ref, b1_ref, w2_ref, b2_ref, w3_ref, b3_ref,
                      out_ref, pool_scr):
    bi = pl.program_id(0)
    y = jnp.dot(a_ref[...], x_ref[...], preferred_element_type=jnp.float32)
    z = jnp.dot(y, w_ref[...], preferred_element_type=jnp.float32) \
        + bias_ref[...]
    pool_scr[pl.ds(bi, 1), :] = jnp.max(z, axis=0, keepdims=True)

    @pl.when(bi == _GB - 1)
    def _mlp():
        p = jnp.concatenate([p0_ref[...], p1_ref[...], p2_ref[...],
                             pool_scr[...]], axis=0)
        h = jnp.tanh(jnp.dot(p, w1_ref[...], preferred_element_type=jnp.float32)
                     + b1_ref[...])
        h = jnp.tanh(jnp.dot(h, w2_ref[...], preferred_element_type=jnp.float32)
                     + b2_ref[...])
        out_ref[...] = (jnp.dot(h, w3_ref[...],
                                preferred_element_type=jnp.float32)
                        + b3_ref[...])


_W2D = pl.BlockSpec((_D, _D), lambda b: (0, 0))
_B2D = pl.BlockSpec((1, _D), lambda b: (0, 0))

_gcn_pool = pl.pallas_call(
    _tc_pool_body,
    grid=(_GB,),
    in_specs=[
        pl.BlockSpec((None, _NH, _NH), lambda b: (b, 0, 0)),
        pl.BlockSpec((None, _NH, _D), lambda b: (b, 0, 0)),
        _W2D, _B2D,
    ],
    out_specs=pl.BlockSpec((_GB, _D), lambda b: (0, 0)),
    out_shape=jax.ShapeDtypeStruct((_GB, _D), jnp.float32),
)

_gcn_pool_mlp = pl.pallas_call(
    _tc_pool_mlp_body,
    grid=(_GB,),
    in_specs=[
        pl.BlockSpec((None, _NH, _NH), lambda b: (b, 0, 0)),
        pl.BlockSpec((None, _NH, _D), lambda b: (b, 0, 0)),
        _W2D, _B2D,
        pl.BlockSpec((_GB, _D), lambda b: (0, 0)),
        pl.BlockSpec((_GB, _D), lambda b: (0, 0)),
        pl.BlockSpec((_GB, _D), lambda b: (0, 0)),
        _W2D, _B2D, _W2D, _B2D, _W2D, _B2D,
    ],
    out_specs=pl.BlockSpec((_B, _D), lambda b: (0, 0)),
    out_shape=jax.ShapeDtypeStruct((_B, _D), jnp.float32),
    scratch_shapes=[pltpu.VMEM((_GB, _D), jnp.float32)],
)


def kernel(x, edge_index, batch, batch_size, W, b, W1, b1, W2, b2, W3, b3):
    N, d = x.shape
    start = edge_index[0]
    end = edge_index[1]
    nodes = x.reshape(_B, _NH, d)
    bias2 = b.reshape(1, _D)
    w3p = jnp.pad(W3, ((0, 0), (0, _D - W3.shape[1])))
    b3p = jnp.pad(b3, (0, _D - b3.shape[0])).reshape(1, _D)
    a_chunks = [sc(start, end).reshape(_GB, _NH, _NH)
                for sc in _scatter_chunks]
    pools = [_gcn_pool(a_chunks[i], nodes[i * _GB:(i + 1) * _GB], W, bias2)
             for i in range(3)]
    out_full = _gcn_pool_mlp(a_chunks[3], nodes[3 * _GB:], W, bias2,
                             pools[0], pools[1], pools[2],
                             W1, b1.reshape(1, _D), W2, b2.reshape(1, _D),
                             w3p, b3p)
    return out_full[:, :1]


# speedup vs baseline: 1.1625x; 1.0017x over previous
import functools

import jax
import jax.numpy as jnp
from jax import lax
from jax.experimental import pallas as pl
from jax.experimental.pallas import tpu as pltpu
from jax.experimental.pallas import tpu_sc as plsc

_B = 16
_NH = 1024
_D = 128
_E = 262144
_NC = 2
_NS = 16
_LN = 16
_GB = _B // 4

_EW = _E // _NS
_ROWS = _EW // 128
_GW = _NH * _NH
_HGW = _GW // 2
_NP = _GB * 2 // _NC
_SW = _HGW // _NS
_ZCH = 16384
_PAD = _HGW


def _make_scatter(gbase):
    def _body(start_hbm, end_hbm, a_hbm,
              start_v, end_v, idx_v, pk_v, zf_v, ones_v, smem_s,
              esem, zsem, ssem, csem):
        cid = lax.axis_index("c")
        sid = lax.axis_index("s")
        ebase = sid * _EW

        e1 = pltpu.async_copy(start_hbm.at[pl.ds(ebase, _EW)], start_v, esem)
        e2 = pltpu.async_copy(end_hbm.at[pl.ds(ebase, _EW)], end_v, esem)

        with jax.named_scope("zfill"):
            @plsc.parallel_loop(0, _ZCH, _LN, unroll=8)
            def _zfill(i):
                zf_v[pl.ds(i, _LN)] = jnp.zeros((_LN,), jnp.float32)
            for k in range(128 // _LN):
                ones_v[pl.ds(k * _LN, _LN)] = jnp.ones((_LN,), jnp.float32)

        e1.wait()
        e2.wait()

        with jax.named_scope("pack"):
            @plsc.parallel_loop(0, _EW, _LN, unroll=8)
            def _pk(i):
                s = start_v[pl.ds(i, _LN)]
                e = end_v[pl.ds(i, _LN)]
                ge = lax.shift_right_logical(s, 10)
                local = lax.shift_left(jnp.bitwise_and(s, 1023), 10) \
                    + jnp.bitwise_and(e, 1023)
                pk_v[pl.ds(i, _LN)] = jnp.bitwise_or(lax.shift_left(ge, 20),
                                                     local)

        dummy = _PAD + sid * _LN + lax.iota(jnp.int32, _LN)

        for p in range(_NP):
            g = gbase + cid * (_NP // 2) + (p // 2)
            half = p % 2
            key = g * 2 + half

            zh = []
            with jax.named_scope("copy_wait_zero"):
                for j in range(_SW // _ZCH):
                    if p > 0:
                        pltpu.make_async_copy(
                            smem_s.at[pl.ds(sid * _SW, _ZCH)],
                            a_hbm.at[pl.ds(0, _ZCH)], csem).wait()
                    zh.append(pltpu.async_copy(
                        zf_v,
                        smem_s.at[pl.ds(sid * _SW + j * _ZCH, _ZCH)], zsem))

            with jax.named_scope("idx_compute"):
                @plsc.parallel_loop(0, _EW, _LN, unroll=8)
                def _ib(i):
                    pk = pk_v[pl.ds(i, _LN)]
                    mine = lax.shift_right_logical(pk, 19) == key
                    local = jnp.bitwise_and(pk, _HGW - 1)
                    idx_v[lax.div(i, 128), pl.ds(lax.rem(i, 128), _LN)] = \
                        jnp.where(mine, local, dummy)

            with jax.named_scope("zero_drain"):
                for h in zh:
                    h.wait()
            plsc.subcore_barrier()

            with jax.named_scope("scatter"):
                @plsc.parallel_loop(0, _ROWS, 1, unroll=8)
                def _sb(j):
                    pltpu.async_copy(ones_v, smem_s.at[idx_v.at[j]], ssem)
                pltpu.make_async_copy(a_hbm.at[pl.ds(0, _ZCH)], zf_v,
                                      ssem).wait()
            plsc.subcore_barrier()

            with jax.named_scope("copy_out"):
                for j in range(_SW // _ZCH):
                    pltpu.async_copy(
                        smem_s.at[pl.ds(sid * _SW + j * _ZCH, _ZCH)],
                        a_hbm.at[pl.ds((g - gbase) * _GW + half * _HGW
                                       + sid * _SW + j * _ZCH, _ZCH)], csem)

        with jax.named_scope("final_wait"):
            for j in range(_SW // _ZCH):
                pltpu.make_async_copy(
                    smem_s.at[pl.ds(sid * _SW, _ZCH)],
                    a_hbm.at[pl.ds(0, _ZCH)], csem).wait()

    return functools.partial(
        pl.kernel,
        out_type=jax.ShapeDtypeStruct((_GB * _NH * _NH,), jnp.float32),
        mesh=plsc.VectorSubcoreMesh(core_axis_name="c", subcore_axis_name="s"),
        scratch_types=[
            pltpu.VMEM((_EW,), jnp.int32),
            pltpu.VMEM((_EW,), jnp.int32),
            pltpu.VMEM((_ROWS, 128), jnp.int32),
            pltpu.VMEM((_EW,), jnp.int32),
            pltpu.VMEM((_ZCH,), jnp.float32),
            pltpu.VMEM((128,), jnp.float32),
            pltpu.VMEM_SHARED((_HGW + 512,), jnp.float32),
            pltpu.SemaphoreType.DMA,
            pltpu.SemaphoreType.DMA,
            pltpu.SemaphoreType.DMA,
            pltpu.SemaphoreType.DMA,
        ],
    )(_body)


_scatter_chunks = [_make_scatter(gb) for gb in range(0, _B, _GB)]


def _tc_pool_body(a_ref, x_ref, w_ref, bias_ref, out_ref):
    bi = pl.program_id(0)
    y = jnp.dot(a_ref[...], x_ref[...], preferred_element_type=jnp.float32)
    z = jnp.dot(y, w_ref[...], preferred_element_type=jnp.float32) \
        + bias_ref[...]
    out_ref[pl.ds(bi, 1), :] = jnp.max(z, axis=0, keepdims=True)


def _tc_pool_mlp_body(a_ref, x_ref, w_ref, bias_ref, p0_ref, p1_ref, p2_ref,
                      w1_ref, b1_ref, w2_ref, b2_ref, w3_ref, b3_ref,
                      out_ref, pool_scr):
    bi = pl.program_id(0)
    y = jnp.dot(a_ref[...], x_ref[...], preferred_element_type=jnp.float32)
    z = jnp.dot(y, w_ref[...], preferred_element_type=jnp.float32) \
        + bias_ref[...]
    pool_scr[pl.ds(bi, 1), :] = jnp.max(z, axis=0, keepdims=True)

    @pl.when(bi == _GB - 1)
    def _mlp():
        p = jnp.concatenate([p0_ref[...], p1_ref[...], p2_ref[...],
                             pool_scr[...]], axis=0)
        h = jnp.tanh(jnp.dot(p, w1_ref[...], preferred_element_type=jnp.float32)
                     + b1_ref[...])
        h = jnp.tanh(jnp.dot(h, w2_ref[...], preferred_element_type=jnp.float32)
                     + b2_ref[...])
        out_ref[...] = (jnp.dot(h, w3_ref[...],
                                preferred_element_type=jnp.float32)
                        + b3_ref[...])


_W2D = pl.BlockSpec((_D, _D), lambda b: (0, 0))
_B2D = pl.BlockSpec((1, _D), lambda b: (0, 0))

_gcn_pool = pl.pallas_call(
    _tc_pool_body,
    grid=(_GB,),
    in_specs=[
        pl.BlockSpec((None, _NH, _NH), lambda b: (b, 0, 0)),
        pl.BlockSpec((None, _NH, _D), lambda b: (b, 0, 0)),
        _W2D, _B2D,
    ],
    out_specs=pl.BlockSpec((_GB, _D), lambda b: (0, 0)),
    out_shape=jax.ShapeDtypeStruct((_GB, _D), jnp.float32),
)

_gcn_pool_mlp = pl.pallas_call(
    _tc_pool_mlp_body,
    grid=(_GB,),
    in_specs=[
        pl.BlockSpec((None, _NH, _NH), lambda b: (b, 0, 0)),
        pl.BlockSpec((None, _NH, _D), lambda b: (b, 0, 0)),
        _W2D, _B2D,
        pl.BlockSpec((_GB, _D), lambda b: (0, 0)),
        pl.BlockSpec((_GB, _D), lambda b: (0, 0)),
        pl.BlockSpec((_GB, _D), lambda b: (0, 0)),
        _W2D, _B2D, _W2D, _B2D, _W2D, _B2D,
    ],
    out_specs=pl.BlockSpec((_B, _D), lambda b: (0, 0)),
    out_shape=jax.ShapeDtypeStruct((_B, _D), jnp.float32),
    scratch_shapes=[pltpu.VMEM((_GB, _D), jnp.float32)],
)


def kernel(x, edge_index, batch, batch_size, W, b, W1, b1, W2, b2, W3, b3):
    N, d = x.shape
    start = edge_index[0]
    end = edge_index[1]
    nodes = x.reshape(_B, _NH, d)
    bias2 = b.reshape(1, _D)
    w3p = jnp.pad(W3, ((0, 0), (0, _D - W3.shape[1])))
    b3p = jnp.pad(b3, (0, _D - b3.shape[0])).reshape(1, _D)
    a_chunks = [sc(start, end).reshape(_GB, _NH, _NH)
                for sc in _scatter_chunks]
    pools = [_gcn_pool(a_chunks[i], nodes[i * _GB:(i + 1) * _GB], W, bias2)
             for i in range(3)]
    out_full = _gcn_pool_mlp(a_chunks[3], nodes[3 * _GB:], W, bias2,
                             pools[0], pools[1], pools[2],
                             W1, b1.reshape(1, _D), W2, b2.reshape(1, _D),
                             w3p, b3p)
    return out_full[:, :1]


# speedup vs baseline: 1.1887x; 1.0225x over previous
import functools

import jax
import jax.numpy as jnp
from jax import lax
from jax.experimental import pallas as pl
from jax.experimental.pallas import tpu as pltpu
from jax.experimental.pallas import tpu_sc as plsc

_B = 16
_NH = 1024
_D = 128
_E = 262144
_NC = 2
_NS = 16
_LN = 16
_GB = _B // 2

_EW = _E // _NS
_ROWS = _EW // 128
_GW = _NH * _NH
_REGW = (_GB // _NC) * _GW
_W = 655360
_NFP = _REGW // _W
_LASTW = _REGW - _NFP * _W
_CH = 16384
_ZCH = 16384


def _wchunks(sz):
    wsz = sz // _NS
    out = []
    for c in (32768, 16384, 8192):
        while wsz >= c:
            out.append(c)
            wsz -= c
    assert wsz == 0
    return out


def _make_scatter(gbase):
    def _body(start_hbm, end_hbm, a_hbm,
              start_v, end_v, idx_v, pk_v, zf_v, ones_v, smem_s,
              esem, zsem, ssem, csem):
        cid = lax.axis_index("c")
        sid = lax.axis_index("s")
        ebase = sid * _EW

        e1 = pltpu.async_copy(start_hbm.at[pl.ds(ebase, _EW)], start_v, esem)
        e2 = pltpu.async_copy(end_hbm.at[pl.ds(ebase, _EW)], end_v, esem)

        with jax.named_scope("zfill"):
            @plsc.parallel_loop(0, _ZCH, _LN, unroll=8)
            def _zfill(i):
                zf_v[pl.ds(i, _LN)] = jnp.zeros((_LN,), jnp.float32)
            for k in range(128 // _LN):
                ones_v[pl.ds(k * _LN, _LN)] = jnp.ones((_LN,), jnp.float32)

        e1.wait()
        e2.wait()

        with jax.named_scope("pack"):
            @plsc.parallel_loop(0, _EW, _LN, unroll=8)
            def _pk(i):
                s = start_v[pl.ds(i, _LN)]
                e = end_v[pl.ds(i, _LN)]
                ge = lax.shift_right_logical(s, 10)
                local = lax.shift_left(jnp.bitwise_and(s, 1023), 10) \
                    + jnp.bitwise_and(e, 1023)
                pk_v[pl.ds(i, _LN)] = jnp.bitwise_or(lax.shift_left(ge, 20),
                                                     local)

        dummy = _W + sid * _LN + lax.iota(jnp.int32, _LN)

        rbase = cid * _REGW
        gword0 = gbase * _GW + rbase

        sizes = [_W] * _NFP + [_LASTW]
        for p, sz in enumerate(sizes):
            lo = gword0 + p * _W
            wsz = sz // _NS

            with jax.named_scope("copy_wait"):
                if p > 0:
                    for c in _wchunks(sizes[p - 1]):
                        pltpu.make_async_copy(
                            smem_s.at[pl.ds(0, c)],
                            a_hbm.at[pl.ds(0, c)], csem).wait()

            with jax.named_scope("zero_fire"):
                zh = []
                off = 0
                for c in _wchunks(sz):
                    for o2 in range(0, c, _ZCH):
                        zh.append(pltpu.async_copy(
                            zf_v.at[pl.ds(0, min(_ZCH, c - o2))],
                            smem_s.at[pl.ds(sid * wsz + off + o2,
                                            min(_ZCH, c - o2))], zsem))
                    off += c

            with jax.named_scope("idx_compute"):
                @plsc.parallel_loop(0, _EW, _LN, unroll=8)
                def _ib(i):
                    pk = pk_v[pl.ds(i, _LN)]
                    mine = jnp.logical_and(pk >= lo, pk < lo + sz)
                    idx_v[lax.div(i, 128), pl.ds(lax.rem(i, 128), _LN)] = \
                        jnp.where(mine, pk - lo, dummy)

            with jax.named_scope("zero_drain"):
                for h in zh:
                    h.wait()
            plsc.subcore_barrier()

            with jax.named_scope("scatter"):
                @plsc.parallel_loop(0, _ROWS, 1, unroll=8)
                def _sb(j):
                    pltpu.async_copy(ones_v, smem_s.at[idx_v.at[j]], ssem)
                pltpu.make_async_copy(a_hbm.at[pl.ds(0, _ZCH)], zf_v,
                                      ssem).wait()
            plsc.subcore_barrier()

            with jax.named_scope("copy_out"):
                off = 0
                for c in _wchunks(sz):
                    pltpu.async_copy(
                        smem_s.at[pl.ds(sid * wsz + off, c)],
                        a_hbm.at[pl.ds(rbase + p * _W + sid * wsz + off, c)],
                        csem)
                    off += c

        with jax.named_scope("final_wait"):
            for c in _wchunks(sizes[-1]):
                pltpu.make_async_copy(
                    smem_s.at[pl.ds(0, c)],
                    a_hbm.at[pl.ds(0, c)], csem).wait()

    return functools.partial(
        pl.kernel,
        out_type=jax.ShapeDtypeStruct((_GB * _NH * _NH,), jnp.float32),
        mesh=plsc.VectorSubcoreMesh(core_axis_name="c", subcore_axis_name="s"),
        scratch_types=[
            pltpu.VMEM((_EW,), jnp.int32),
            pltpu.VMEM((_EW,), jnp.int32),
            pltpu.VMEM((_ROWS, 128), jnp.int32),
            pltpu.VMEM((_EW,), jnp.int32),
            pltpu.VMEM((_ZCH,), jnp.float32),
            pltpu.VMEM((128,), jnp.float32),
            pltpu.VMEM_SHARED((_W + 512,), jnp.float32),
            pltpu.SemaphoreType.DMA,
            pltpu.SemaphoreType.DMA,
            pltpu.SemaphoreType.DMA,
            pltpu.SemaphoreType.DMA,
        ],
    )(_body)


_scatter_lo = _make_scatter(0)
_scatter_hi = _make_scatter(_GB)


def _tc_pool_body(a_ref, x_ref, w_ref, bias_ref, out_ref):
    bi = pl.program_id(0)
    y = jnp.dot(a_ref[...], x_ref[...], preferred_element_type=jnp.float32)
    z = jnp.dot(y, w_ref[...], preferred_element_type=jnp.float32) \
        + bias_ref[...]
    out_ref[pl.ds(bi, 1), :] = jnp.max(z, axis=0, keepdims=True)


def _tc_pool_mlp_body(a_ref, x_ref, w_ref, bias_ref, p0_ref, w1_ref, b1_ref,
                      w2_ref, b2_ref, w3_ref, b3_ref, out_ref, pool_scr):
    bi = pl.program_id(0)
    y = jnp.dot(a_ref[...], x_ref[...], preferred_element_type=jnp.float32)
    z = jnp.dot(y, w_ref[...], preferred_element_type=jnp.float32) \
        + bias_ref[...]
    pool_scr[pl.ds(bi, 1), :] = jnp.max(z, axis=0, keepdims=True)

    @pl.when(bi == _GB - 1)
    def _mlp():
        p = jnp.concatenate([p0_ref[...], pool_scr[...]], axis=0)
        h = jnp.tanh(jnp.dot(p, w1_ref[...], preferred_element_type=jnp.float32)
                     + b1_ref[...])
        h = jnp.tanh(jnp.dot(h, w2_ref[...], preferred_element_type=jnp.float32)
                     + b2_ref[...])
        out_ref[...] = (jnp.dot(h, w3_ref[...],
                                preferred_element_type=jnp.float32)
                        + b3_ref[...])


_W2D = pl.BlockSpec((_D, _D), lambda b: (0, 0))
_B2D = pl.BlockSpec((1, _D), lambda b: (0, 0))

_gcn_pool = pl.pallas_call(
    _tc_pool_body,
    grid=(_GB,),
    in_specs=[
        pl.BlockSpec((None, _NH, _NH), lambda b: (b, 0, 0)),
        pl.BlockSpec((None, _NH, _D), lambda b: (b, 0, 0)),
        _W2D, _B2D,
    ],
    out_specs=pl.BlockSpec((_GB, _D), lambda b: (0, 0)),
    out_shape=jax.ShapeDtypeStruct((_GB, _D), jnp.float32),
)

_gcn_pool_mlp = pl.pallas_call(
    _tc_pool_mlp_body,
    grid=(_GB,),
    in_specs=[
        pl.BlockSpec((None, _NH, _NH), lambda b: (b, 0, 0)),
        pl.BlockSpec((None, _NH, _D), lambda b: (b, 0, 0)),
        _W2D, _B2D,
        pl.BlockSpec((_GB, _D), lambda b: (0, 0)),
        _W2D, _B2D, _W2D, _B2D, _W2D, _B2D,
    ],
    out_specs=pl.BlockSpec((_B, _D), lambda b: (0, 0)),
    out_shape=jax.ShapeDtypeStruct((_B, _D), jnp.float32),
    scratch_shapes=[pltpu.VMEM((_GB, _D), jnp.float32)],
)


def kernel(x, edge_index, batch, batch_size, W, b, W1, b1, W2, b2, W3, b3):
    N, d = x.shape
    start = edge_index[0]
    end = edge_index[1]
    a_lo = _scatter_lo(start, end).reshape(_GB, _NH, _NH)
    a_hi = _scatter_hi(start, end).reshape(_GB, _NH, _NH)
    nodes = x.reshape(_B, _NH, d)
    bias2 = b.reshape(1, _D)
    w3p = jnp.pad(W3, ((0, 0), (0, _D - W3.shape[1])))
    b3p = jnp.pad(b3, (0, _D - b3.shape[0])).reshape(1, _D)
    pooled_lo = _gcn_pool(a_lo, nodes[:_GB], W, bias2)
    out_full = _gcn_pool_mlp(a_hi, nodes[_GB:], W, bias2, pooled_lo,
                             W1, b1.reshape(1, _D), W2, b2.reshape(1, _D),
                             w3p, b3p)
    return out_full[:, :1]
